# baseline (speedup 1.0000x reference)
import functools
import jax
import jax.numpy as jnp
from jax import lax
from jax.experimental import pallas as pl
from jax.experimental.pallas import tpu as pltpu
from jax.experimental.pallas import tpu_sc as plsc

NC = 2
NS = 16
NW = NC * NS

D = 64
WPA = 6
AV_PAD = 1024
APW = AV_PAD // NW
IPW = APW * WPA

_mesh = plsc.VectorSubcoreMesh(core_axis_name="c", subcore_axis_name="s")
_params = pltpu.CompilerParams(use_tc_tiling_on_sc=False, needs_layout_passes=False)


def _wid():
    return lax.axis_index("s") * NC + lax.axis_index("c")


@functools.partial(
    pl.kernel,
    out_type=jax.ShapeDtypeStruct((D, AV_PAD), jnp.float32),
    mesh=_mesh,
    scratch_types=[
        pltpu.VMEM((IPW,), jnp.int32),
        pltpu.VMEM((IPW, D), jnp.float32),
        pltpu.VMEM((D, APW), jnp.float32),
        pltpu.SemaphoreType.DMA,
    ],
    compiler_params=_params,
)
def _build_table(a2w_hbm, emb_hbm, table_hbm, idx_v, rows_v, out_v, sem):
    wid = _wid()
    base = wid * IPW
    pltpu.sync_copy(a2w_hbm.at[pl.ds(base, IPW)], idx_v)
    h = IPW // 2
    pltpu.async_copy(emb_hbm.at[idx_v.at[pl.ds(0, h)]],
                     rows_v.at[pl.ds(0, h)], sem).wait()
    pltpu.async_copy(emb_hbm.at[idx_v.at[pl.ds(h, h)]],
                     rows_v.at[pl.ds(h, h)], sem).wait()
    iota16 = lax.iota(jnp.int32, 16)
    for j in range(APW):
        for c in range(D // 16):
            s = pl.ds(16 * c, 16)
            acc = rows_v[WPA * j, s]
            for k in range(1, WPA):
                acc = acc + rows_v[WPA * j + k, s]
            plsc.store_scatter(out_v, [iota16 + (16 * c),
                                       jnp.full((16,), j, jnp.int32)], acc)
    pltpu.sync_copy(out_v, table_hbm.at[:, pl.ds(wid * APW, APW)])


BATCH = 16384
SEQ = 50
BPW = BATCH // NW
NTB = BPW // 128
NUNIT = SEQ * NTB


@functools.partial(
    pl.kernel,
    out_type=jax.ShapeDtypeStruct((SEQ, D, BATCH), jnp.float32),
    mesh=_mesh,
    scratch_types=[
        pltpu.VMEM((BPW, SEQ), jnp.int32),
        pltpu.VMEM((D * AV_PAD,), jnp.float32),
        [pltpu.VMEM((D, 128), jnp.float32) for _ in range(2)],
        [pltpu.SemaphoreType.DMA for _ in range(2)],
    ],
    compiler_params=_params,
)
def _lookup(x_hbm, tt_hbm, out_hbm, x_v, tt_v, sbufs, osems):
    wid = _wid()
    b0 = wid * BPW
    for d in range(D):
        pltpu.sync_copy(tt_hbm.at[d], tt_v.at[pl.ds(d * AV_PAD, AV_PAD)])
    pltpu.sync_copy(x_hbm.at[pl.ds(b0, BPW), :], x_v)
    iota16 = lax.iota(jnp.int32, 16)

    def unit(i, p):
        s = i // NTB
        t = i % NTB
        sb = sbufs[p]
        scol = jnp.full((16,), 0, jnp.int32) + s
        cur = [
            plsc.load_gather(x_v, [iota16 + (t * 128 + 16 * c), scol])
            for c in range(8)
        ]
        for d in range(D):
            for c in range(8):
                sb[d, pl.ds(16 * c, 16)] = plsc.load_gather(tt_v, [cur[c]])
            if d + 1 < D:
                cur = [v + AV_PAD for v in cur]
        pltpu.async_copy(
            sb, out_hbm.at[s, :, pl.ds(b0 + t * 128, 128)], osems[p])

    def body(g, carry):
        for p in range(2):
            @pl.when(g > 0)
            def _():
                pltpu.make_async_copy(
                    sbufs[p], out_hbm.at[0, :, pl.ds(0, 128)], osems[p]).wait()

            unit(2 * g + p, p)
        return carry

    lax.fori_loop(0, NUNIT // 2, body, 0)
    for p in range(2):
        pltpu.make_async_copy(
            sbufs[p], out_hbm.at[0, :, pl.ds(0, 128)], osems[p]).wait()


def kernel(x, action_to_words, word_embedding):
    b, s = x.shape
    a2w_flat = jnp.pad(action_to_words.reshape(-1),
                       (0, AV_PAD * WPA - action_to_words.size))
    table_t = _build_table(a2w_flat, word_embedding)
    out_t = _lookup(x, table_t)
    return jnp.transpose(out_t, (2, 0, 1))


# speedup vs baseline: 1.5739x; 1.5739x over previous
import functools
import jax
import jax.numpy as jnp
from jax import lax
from jax.experimental import pallas as pl
from jax.experimental.pallas import tpu as pltpu
from jax.experimental.pallas import tpu_sc as plsc

NC = 2
NS = 16
NW = NC * NS

D = 64
WPA = 6
AV_PAD = 1024
APW = AV_PAD // NW
IPW = APW * WPA

_mesh = plsc.VectorSubcoreMesh(core_axis_name="c", subcore_axis_name="s")
_params = pltpu.CompilerParams(use_tc_tiling_on_sc=False, needs_layout_passes=False)


def _wid():
    return lax.axis_index("s") * NC + lax.axis_index("c")


@functools.partial(
    pl.kernel,
    out_type=jax.ShapeDtypeStruct((D, AV_PAD), jnp.float32),
    mesh=_mesh,
    scratch_types=[
        pltpu.VMEM((IPW,), jnp.int32),
        pltpu.VMEM((IPW, D), jnp.float32),
        pltpu.VMEM((D, APW), jnp.float32),
        pltpu.SemaphoreType.DMA,
    ],
    compiler_params=_params,
)
def _build_table(a2w_hbm, emb_hbm, table_hbm, idx_v, rows_v, out_v, sem):
    wid = _wid()
    base = wid * IPW
    pltpu.sync_copy(a2w_hbm.at[pl.ds(base, IPW)], idx_v)
    h = IPW // 2
    pltpu.async_copy(emb_hbm.at[idx_v.at[pl.ds(0, h)]],
                     rows_v.at[pl.ds(0, h)], sem).wait()
    pltpu.async_copy(emb_hbm.at[idx_v.at[pl.ds(h, h)]],
                     rows_v.at[pl.ds(h, h)], sem).wait()
    iota16 = lax.iota(jnp.int32, 16)
    for j in range(APW):
        for c in range(D // 16):
            s = pl.ds(16 * c, 16)
            acc = rows_v[WPA * j, s]
            for k in range(1, WPA):
                acc = acc + rows_v[WPA * j + k, s]
            plsc.store_scatter(out_v, [iota16 + (16 * c),
                                       jnp.full((16,), j, jnp.int32)], acc)
    pltpu.sync_copy(out_v, table_hbm.at[:, pl.ds(wid * APW, APW)])


BATCH = 16384
SEQ = 50
BPW = BATCH // NW
NTB = BPW // 128
NUNIT = SEQ * NTB


@functools.partial(
    pl.kernel,
    out_type=jax.ShapeDtypeStruct((SEQ, D, BATCH), jnp.float32),
    mesh=_mesh,
    scratch_types=[
        pltpu.VMEM((BPW, SEQ), jnp.int32),
        pltpu.VMEM((D * AV_PAD,), jnp.float32),
        [pltpu.VMEM((D, 128), jnp.float32) for _ in range(2)],
        [pltpu.SemaphoreType.DMA for _ in range(2)],
    ],
    compiler_params=_params,
)
def _lookup(x_hbm, tt_hbm, out_hbm, x_v, tt_v, sbufs, osems):
    wid = _wid()
    b0 = wid * BPW
    for d in range(D):
        pltpu.sync_copy(tt_hbm.at[d], tt_v.at[pl.ds(d * AV_PAD, AV_PAD)])
    pltpu.sync_copy(x_hbm.at[pl.ds(b0, BPW), :], x_v)
    iota16 = lax.iota(jnp.int32, 16)

    def unit(i, p):
        s = i // NTB
        t = i % NTB
        sb = sbufs[p]
        scol = jnp.full((16,), 0, jnp.int32) + s
        cur = [
            plsc.load_gather(x_v, [iota16 + (t * 128 + 16 * c), scol])
            for c in range(8)
        ]
        for d in range(D):
            vals = [plsc.load_gather(tt_v, [cur[c]]) for c in range(8)]
            if d + 1 < D:
                cur = [v + AV_PAD for v in cur]
            for c in range(8):
                sb[d, pl.ds(16 * c, 16)] = vals[c]
        pltpu.async_copy(
            sb, out_hbm.at[s, :, pl.ds(b0 + t * 128, 128)], osems[p])

    def body(g, carry):
        for p in range(2):
            @pl.when(g > 0)
            def _():
                pltpu.make_async_copy(
                    sbufs[p], out_hbm.at[0, :, pl.ds(0, 128)], osems[p]).wait()

            unit(2 * g + p, p)
        return carry

    lax.fori_loop(0, NUNIT // 2, body, 0)
    for p in range(2):
        pltpu.make_async_copy(
            sbufs[p], out_hbm.at[0, :, pl.ds(0, 128)], osems[p]).wait()


def kernel(x, action_to_words, word_embedding):
    b, s = x.shape
    a2w_flat = jnp.pad(action_to_words.reshape(-1),
                       (0, AV_PAD * WPA - action_to_words.size))
    table_t = _build_table(a2w_flat, word_embedding)
    out_t = _lookup(x, table_t)
    return jnp.transpose(out_t, (2, 0, 1))
